# baseline (speedup 1.0000x reference)
import functools

import jax
import jax.numpy as jnp
from jax import lax
from jax.experimental import pallas as pl
from jax.experimental.pallas import tpu as pltpu
from jax.experimental.pallas import tpu_sc as plsc

_D = 5
_LANES = 128
_NC = 2
_NS = 16
_NW = _NC * _NS
_B = 16384
_IDXROWS = _D * _B // _LANES
_IPW = _IDXROWS // _NW

_M_CF = 8


@functools.cache
def _sc_gather_fn():
    mesh = plsc.VectorSubcoreMesh(core_axis_name="c", subcore_axis_name="s")

    @functools.partial(
        pl.kernel,
        mesh=mesh,
        out_type=(
            jax.ShapeDtypeStruct((_IDXROWS, _LANES), jnp.float32),
            jax.ShapeDtypeStruct((_IDXROWS, _LANES), jnp.float32),
        ),
        scratch_types=[
            pltpu.VMEM((_IPW, _LANES), jnp.int32),
            pltpu.VMEM((_IPW, _LANES), jnp.int32),
            pltpu.VMEM((_IPW, _LANES), jnp.float32),
            pltpu.VMEM((_IPW, _LANES), jnp.float32),
            pltpu.SemaphoreType.DMA,
            pltpu.SemaphoreType.DMA,
        ],
        compiler_params=pltpu.CompilerParams(use_tc_tiling_on_sc=False),
    )
    def _sc_gather(uidx_hbm, iidx_hbm, uw_hbm, iw_hbm, out_u, out_i,
                   uidx_v, iidx_v, uval_v, ival_v, usem, isem):
        wid = lax.axis_index("s") * _NC + lax.axis_index("c")
        r0 = wid * _IPW
        pltpu.sync_copy(uidx_hbm.at[pl.ds(r0, _IPW)], uidx_v)
        pltpu.sync_copy(iidx_hbm.at[pl.ds(r0, _IPW)], iidx_v)
        cps = []
        for j in range(_IPW):
            cps.append(pltpu.async_copy(uw_hbm.at[uidx_v.at[j]], uval_v.at[j], usem))
            cps.append(pltpu.async_copy(iw_hbm.at[iidx_v.at[j]], ival_v.at[j], isem))
        for c in cps:
            c.wait()
        pltpu.sync_copy(uval_v, out_u.at[pl.ds(r0, _IPW)])
        pltpu.sync_copy(ival_v, out_i.at[pl.ds(r0, _IPW)])

    return _sc_gather


def _gammln(x):
    cof = (76.18009172947146, -86.50532032941677, 24.01409824083091,
           -1.231739572450155, 0.1208650973866179e-2, -0.5395239384953e-5)
    tmp = x + 5.5
    tmp = tmp - (x + 0.5) * jnp.log(tmp)
    ser = 1.000000000190015
    y = x
    for c in cof:
        y = y + 1.0
        ser = ser + c / y
    return -tmp + jnp.log(2.5066282746310005 * ser / x)


def _tc_math(u_ref, i_ref, a_ref, b_ref, mass_ref, edges_ref):
    s = u_ref[...] + i_ref[...]
    ui = jnp.exp(s)
    tot = jnp.sum(ui, axis=0, keepdims=True)
    un = ui / tot
    e = [un[0:1]]
    for j in range(1, _D):
        e.append(e[-1] + un[j:j + 1])
    edges_ref[...] = jnp.concatenate(e, axis=0)

    a1 = a_ref[...]
    b1 = b_ref[...]
    lnbeta = _gammln(a1) + _gammln(b1) - _gammln(a1 + b1)

    m = _D - 1
    bn = a1.shape[1]
    x = jnp.concatenate(e[:m], axis=0)
    a = jnp.broadcast_to(a1, (m, bn))
    b = jnp.broadcast_to(b1, (m, bn))
    lb = jnp.broadcast_to(lnbeta, (m, bn))

    flip = x > (a + 1.0) / (a + b + 2.0)
    xx = jnp.where(flip, 1.0 - x, x)
    aa = jnp.where(flip, b, a)
    bb = jnp.where(flip, a, b)
    lnpre = aa * jnp.log(xx) + bb * jnp.log(1.0 - xx) - lb
    front = jnp.exp(lnpre) / aa

    tiny = 1e-30
    qab = aa + bb
    qap = aa + 1.0
    qam = aa - 1.0
    c = jnp.ones_like(xx)
    d = 1.0 - qab * xx / qap
    d = jnp.where(jnp.abs(d) < tiny, tiny, d)
    d = 1.0 / d
    h = d
    for mm in range(1, _M_CF + 1):
        m2 = 2.0 * mm
        num = mm * (bb - mm) * xx / ((qam + m2) * (aa + m2))
        d = 1.0 + num * d
        d = jnp.where(jnp.abs(d) < tiny, tiny, d)
        c = 1.0 + num / c
        c = jnp.where(jnp.abs(c) < tiny, tiny, c)
        d = 1.0 / d
        h = h * d * c
        num = -(aa + mm) * (qab + mm) * xx / ((aa + m2) * (qap + m2))
        d = 1.0 + num * d
        d = jnp.where(jnp.abs(d) < tiny, tiny, d)
        c = 1.0 + num / c
        c = jnp.where(jnp.abs(c) < tiny, tiny, c)
        d = 1.0 / d
        h = h * d * c
    cdf = front * h
    cdf = jnp.where(flip, 1.0 - cdf, cdf)
    cdf = jnp.clip(cdf, 0.0, 1.0)

    mlist = [cdf[0:1]]
    for j in range(1, m):
        mlist.append(cdf[j:j + 1] - cdf[j - 1:j])
    mlist.append(1.0 - cdf[m - 1:m])
    mass_ref[...] = jnp.concatenate(mlist, axis=0)


def _tc_call(uT, iT, aT, bT):
    bn = uT.shape[1]
    return pl.pallas_call(
        _tc_math,
        out_shape=(
            jax.ShapeDtypeStruct((_D, bn), jnp.float32),
            jax.ShapeDtypeStruct((_D, bn), jnp.float32),
        ),
    )(uT, iT, aT, bT)


def kernel(uid, iid, alpha, beta, uid_w, iid_w):
    b = uid.shape[0]
    offs = jnp.arange(_D, dtype=jnp.int32)[:, None]
    uwidx = (uid.astype(jnp.int32)[None, :] * 8 + offs).reshape(_IDXROWS, _LANES)
    iwidx = (iid.astype(jnp.int32)[None, :] * 8 + offs).reshape(_IDXROWS, _LANES)
    upad = jnp.pad(uid_w, ((0, 0), (0, 3))).reshape(-1)
    ipad = jnp.pad(iid_w, ((0, 0), (0, 3))).reshape(-1)
    gu, gi = _sc_gather_fn()(uwidx, iwidx, upad, ipad)
    uT = gu.reshape(_D, b)
    iT = gi.reshape(_D, b)
    aT = alpha.reshape(1, b)
    bT = beta.reshape(1, b)
    massT, edgesT = _tc_call(uT, iT, aT, bT)
    return massT.T, edgesT.T


# speedup vs baseline: 1.5654x; 1.5654x over previous
import functools

import jax
import jax.numpy as jnp
from jax import lax
from jax.experimental import pallas as pl
from jax.experimental.pallas import tpu as pltpu
from jax.experimental.pallas import tpu_sc as plsc

_D = 5
_LANES = 128
_NC = 2
_NS = 16
_NW = _NC * _NS
_B = 16384
_IDXROWS = _D * _B // _LANES
_IPW = _IDXROWS // _NW

_M_CF = 8


@functools.cache
def _sc_gather_fn():
    mesh = plsc.VectorSubcoreMesh(core_axis_name="c", subcore_axis_name="s")

    @functools.partial(
        pl.kernel,
        mesh=mesh,
        out_type=(
            jax.ShapeDtypeStruct((_IDXROWS, _LANES), jnp.float32),
            jax.ShapeDtypeStruct((_IDXROWS, _LANES), jnp.float32),
        ),
        scratch_types=[
            pltpu.VMEM((_IPW, _LANES), jnp.int32),
            pltpu.VMEM((_IPW, _LANES), jnp.int32),
            pltpu.VMEM((_IPW, _LANES), jnp.float32),
            pltpu.VMEM((_IPW, _LANES), jnp.float32),
            pltpu.SemaphoreType.DMA,
            pltpu.SemaphoreType.DMA,
        ],
        compiler_params=pltpu.CompilerParams(use_tc_tiling_on_sc=False),
    )
    def _sc_gather(uidx_hbm, iidx_hbm, uw_hbm, iw_hbm, out_u, out_i,
                   uidx_v, iidx_v, uval_v, ival_v, usem, isem):
        wid = lax.axis_index("s") * _NC + lax.axis_index("c")
        r0 = wid * _IPW
        pltpu.sync_copy(uidx_hbm.at[pl.ds(r0, _IPW)], uidx_v)
        pltpu.sync_copy(iidx_hbm.at[pl.ds(r0, _IPW)], iidx_v)
        cps = []
        for j in range(_IPW):
            cps.append(pltpu.async_copy(uw_hbm.at[uidx_v.at[j]], uval_v.at[j], usem))
            cps.append(pltpu.async_copy(iw_hbm.at[iidx_v.at[j]], ival_v.at[j], isem))
        for c in cps:
            c.wait()
        pltpu.sync_copy(uval_v, out_u.at[pl.ds(r0, _IPW)])
        pltpu.sync_copy(ival_v, out_i.at[pl.ds(r0, _IPW)])

    return _sc_gather


def _gammln(x):
    cof = (76.18009172947146, -86.50532032941677, 24.01409824083091,
           -1.231739572450155, 0.1208650973866179e-2, -0.5395239384953e-5)
    tmp = x + 5.5
    tmp = tmp - (x + 0.5) * jnp.log(tmp)
    ser = 1.000000000190015
    y = x
    for c in cof:
        y = y + 1.0
        ser = ser + c / y
    return -tmp + jnp.log(2.5066282746310005 * ser / x)


def _tc_math(u_ref, i_ref, a_ref, b_ref, mass_ref, edges_ref):
    s = u_ref[...] + i_ref[...]
    ui = jnp.exp(s)
    tot = jnp.sum(ui, axis=0, keepdims=True)
    un = ui / tot
    e = [un[0:1]]
    for j in range(1, _D):
        e.append(e[-1] + un[j:j + 1])
    edges_ref[...] = jnp.concatenate(e, axis=0)

    a1 = a_ref[...]
    b1 = b_ref[...]
    lnbeta = _gammln(a1) + _gammln(b1) - _gammln(a1 + b1)

    m = _D - 1
    bn = a1.shape[1]
    x = jnp.concatenate(e[:m], axis=0)
    a = jnp.broadcast_to(a1, (m, bn))
    b = jnp.broadcast_to(b1, (m, bn))
    lb = jnp.broadcast_to(lnbeta, (m, bn))

    flip = x > (a + 1.0) / (a + b + 2.0)
    xx = jnp.where(flip, 1.0 - x, x)
    aa = jnp.where(flip, b, a)
    bb = jnp.where(flip, a, b)
    lnpre = aa * jnp.log(xx) + bb * jnp.log(1.0 - xx) - lb
    front = jnp.exp(lnpre) / aa

    tiny = 1e-30
    qab = aa + bb
    qap = aa + 1.0
    qam = aa - 1.0
    c = jnp.ones_like(xx)
    d = 1.0 - qab * xx / qap
    d = jnp.where(jnp.abs(d) < tiny, tiny, d)
    d = 1.0 / d
    h = d
    for mm in range(1, _M_CF + 1):
        m2 = 2.0 * mm
        num = mm * (bb - mm) * xx / ((qam + m2) * (aa + m2))
        d = 1.0 + num * d
        d = jnp.where(jnp.abs(d) < tiny, tiny, d)
        c = 1.0 + num / c
        c = jnp.where(jnp.abs(c) < tiny, tiny, c)
        d = 1.0 / d
        h = h * d * c
        num = -(aa + mm) * (qab + mm) * xx / ((aa + m2) * (qap + m2))
        d = 1.0 + num * d
        d = jnp.where(jnp.abs(d) < tiny, tiny, d)
        c = 1.0 + num / c
        c = jnp.where(jnp.abs(c) < tiny, tiny, c)
        d = 1.0 / d
        h = h * d * c
    cdf = front * h
    cdf = jnp.where(flip, 1.0 - cdf, cdf)
    cdf = jnp.clip(cdf, 0.0, 1.0)

    mlist = [cdf[0:1]]
    for j in range(1, m):
        mlist.append(cdf[j:j + 1] - cdf[j - 1:j])
    mlist.append(1.0 - cdf[m - 1:m])
    mass_ref[...] = jnp.concatenate(mlist, axis=0)


def _tc_call(uT, iT, aT, bT):
    bn = uT.shape[1]
    return pl.pallas_call(
        _tc_math,
        out_shape=(
            jax.ShapeDtypeStruct((_D, bn), jnp.float32),
            jax.ShapeDtypeStruct((_D, bn), jnp.float32),
        ),
    )(uT, iT, aT, bT)


def kernel(uid, iid, alpha, beta, uid_w, iid_w):
    b = uid.shape[0]
    offs = jnp.arange(_D, dtype=jnp.int32)[:, None]
    uwidx = (uid.astype(jnp.int32)[None, :] * _D + offs).reshape(_IDXROWS, _LANES)
    iwidx = (iid.astype(jnp.int32)[None, :] * _D + offs).reshape(_IDXROWS, _LANES)
    gu, gi = _sc_gather_fn()(uwidx, iwidx, uid_w.reshape(-1), iid_w.reshape(-1))
    uT = gu.reshape(_D, b)
    iT = gi.reshape(_D, b)
    aT = alpha.reshape(1, b)
    bT = beta.reshape(1, b)
    massT, edgesT = _tc_call(uT, iT, aT, bT)
    return massT.T, edgesT.T
